# baseline (speedup 1.0000x reference)
import functools

import jax
import jax.numpy as jnp
from jax import lax
from jax.experimental import pallas as pl
from jax.experimental.pallas import tpu as pltpu
from jax.experimental.pallas import tpu_sc as plsc

N = 10000
E = 320000
F = 128
D = 32
C = 16

NW = 32
CHUNK = 128
NCHT = E // CHUNK
T1 = 1240
NCH0 = 79
NCH1 = 78
NPAD = 10112
RPT = NPAD // 16


RING = 6


def _seg_kernel_body(y_hbm, edge_hbm, out_hbm,
                     src_v, dst_v, rows, stage, acc, sem, scsem, sem_i):
    c = lax.axis_index("c")
    s = lax.axis_index("s")
    base = jnp.where(c == 0, T1 + s * 78 + jnp.maximum(s - 4, 0),
                     s * 77 + jnp.maximum(s - 8, 0))
    nch = jnp.where(c == 0, 78 + (s >= 4).astype(jnp.int32),
                    77 + (s >= 8).astype(jnp.int32))

    @pl.when(c == 0)
    def _():
        pltpu.async_copy(edge_hbm.at[0, pl.ds(base, NCH0)],
                         src_v.at[pl.ds(0, NCH0)], sem_i)
        pltpu.async_copy(edge_hbm.at[1, pl.ds(base, NCH0)],
                         dst_v.at[pl.ds(0, NCH0)], sem_i)

    @pl.when(c == 1)
    def _():
        pltpu.async_copy(edge_hbm.at[0, pl.ds(base, NCH1)],
                         src_v.at[pl.ds(0, NCH1)], sem_i)
        pltpu.async_copy(edge_hbm.at[1, pl.ds(base, NCH1)],
                         dst_v.at[pl.ds(0, NCH1)], sem_i)

    zero = jnp.zeros((16,), jnp.float32)

    def zrow(i, _):
        stage[i, pl.ds(0, 16)] = zero
        stage[i, pl.ds(16, 16)] = zero
        return 0

    lax.fori_loop(0, RPT, zrow, 0)
    pltpu.sync_copy(stage, acc.at[pl.ds(s * RPT, RPT)])

    @pl.when(c == 0)
    def _():
        pltpu.make_async_copy(edge_hbm.at[0, pl.ds(0, NCH0)],
                              src_v.at[pl.ds(0, NCH0)], sem_i).wait()
        pltpu.make_async_copy(edge_hbm.at[1, pl.ds(0, NCH0)],
                              dst_v.at[pl.ds(0, NCH0)], sem_i).wait()

    @pl.when(c == 1)
    def _():
        pltpu.make_async_copy(edge_hbm.at[0, pl.ds(0, NCH1)],
                              src_v.at[pl.ds(0, NCH1)], sem_i).wait()
        pltpu.make_async_copy(edge_hbm.at[1, pl.ds(0, NCH1)],
                              dst_v.at[pl.ds(0, NCH1)], sem_i).wait()

    plsc.subcore_barrier()

    for k in range(RING - 1):
        @pl.when(nch > k)
        def _(k=k):
            pltpu.async_copy(y_hbm.at[src_v.at[k]], rows.at[k], sem.at[k])

    def step(j, _):
        m = j % RING

        @pl.when(j + RING - 1 < nch)
        def _():
            m2 = (j + RING - 1) % RING

            @pl.when(j >= 1)
            def _():
                pltpu.make_async_copy(rows.at[m2], acc.at[dst_v.at[0]],
                                      scsem.at[m2]).wait()

            pltpu.async_copy(y_hbm.at[src_v.at[j + RING - 1]],
                             rows.at[m2], sem.at[m2])

        pltpu.make_async_copy(y_hbm.at[src_v.at[0]], rows.at[m], sem.at[m]).wait()
        pltpu.async_copy(rows.at[m], acc.at[dst_v.at[j]], scsem.at[m], add=True)
        return 0

    lax.fori_loop(0, nch, step, 0)

    for k in range(RING):
        pltpu.make_async_copy(rows.at[k], acc.at[dst_v.at[0]],
                              scsem.at[k]).wait()
    plsc.subcore_barrier()

    pltpu.sync_copy(acc.at[pl.ds(s * RPT, RPT)], stage)
    pltpu.sync_copy(stage, out_hbm.at[c, pl.ds(s * RPT, RPT)])


_seg_sum = pl.kernel(
    _seg_kernel_body,
    out_type=jax.ShapeDtypeStruct((2, NPAD, D), jnp.float32),
    mesh=plsc.VectorSubcoreMesh(core_axis_name="c", subcore_axis_name="s"),
    scratch_types=[
        pltpu.VMEM((NCH0, CHUNK), jnp.int32),
        pltpu.VMEM((NCH0, CHUNK), jnp.int32),
        pltpu.VMEM((RING, CHUNK, D), jnp.float32),
        pltpu.VMEM((RPT, D), jnp.float32),
        pltpu.VMEM_SHARED((NPAD, D), jnp.float32),
        pltpu.SemaphoreType.DMA((RING,)),
        pltpu.SemaphoreType.DMA((RING,)),
        pltpu.SemaphoreType.DMA,
    ],
    compiler_params=pltpu.CompilerParams(use_tc_tiling_on_sc=False),
)


NR = N // 4


def _mm_body(x_ref, w_ref, o_ref):
    for g in range(4):
        o_ref[:, D * g:D * (g + 1)] = jnp.dot(
            x_ref[NR * g:NR * (g + 1), :], w_ref[...],
            preferred_element_type=jnp.float32)


_proj = pl.pallas_call(
    _mm_body,
    out_shape=jax.ShapeDtypeStruct((NR, 4 * D), jnp.float32),
)


def _fold_stats(r):
    cs = jnp.sum(r, axis=0, keepdims=True)
    css = jnp.sum(r * r, axis=0, keepdims=True)
    s32 = cs[:, 0:32] + cs[:, 32:64] + cs[:, 64:96] + cs[:, 96:128]
    ss32 = css[:, 0:32] + css[:, 32:64] + css[:, 64:96] + css[:, 96:128]
    m = jnp.concatenate([s32] * 4, axis=1) * (1.0 / N)
    ms2 = jnp.concatenate([ss32] * 4, axis=1) * (1.0 / N)
    return m, ms2


def _bn128(r, g, b):
    m, ms2 = _fold_stats(r)
    v = ms2 - m * m
    return (r - m) * jax.lax.rsqrt(v + 1e-5) * g + b


def _gin128(y, parts, b_a, w_b, b_b):
    agg = parts[0, :NR, :] + parts[1, :NR, :]
    z = jnp.maximum(y + agg + b_a, 0.0)
    return jnp.dot(z, w_b, preferred_element_type=jnp.float32) + b_b


def _mid_body(y_ref, parts_ref, b1_ref, w2_ref, b2_ref, g1_ref, be1_ref,
              w3_ref, o_ref):
    h = jnp.maximum(_gin128(y_ref[...], parts_ref[...], b1_ref[...],
                            w2_ref[...], b2_ref[...]), 0.0)
    h = _bn128(h, g1_ref[...], be1_ref[...])
    o_ref[...] = jnp.dot(h, w3_ref[...], preferred_element_type=jnp.float32)


_mid = pl.pallas_call(
    _mid_body,
    out_shape=jax.ShapeDtypeStruct((NR, 4 * D), jnp.float32),
)


def _tail_body(y_ref, parts_ref, b3_ref, w4_ref, b4_ref, g2_ref, be2_ref,
               wf1_ref, bf1_ref, wf2_ref, bf2_ref, o_ref):
    h = jnp.maximum(_gin128(y_ref[...], parts_ref[...], b3_ref[...],
                            w4_ref[...], b4_ref[...]), 0.0)
    h = _bn128(h, g2_ref[...], be2_ref[...])
    f = jnp.maximum(
        jnp.dot(h, wf1_ref[...], preferred_element_type=jnp.float32)
        + bf1_ref[...], 0.0)
    logits = jnp.dot(f, wf2_ref[...], preferred_element_type=jnp.float32) \
        + bf2_ref[...]
    r64 = jax.lax.broadcasted_iota(jnp.int32, (4 * C, 4 * C), 0) // C
    c64 = jax.lax.broadcasted_iota(jnp.int32, (4 * C, 4 * C), 1) // C
    ones_blk = (r64 == c64).astype(jnp.float32)
    se = jnp.dot(jnp.exp(logits), ones_blk, preferred_element_type=jnp.float32)
    out = logits - jnp.log(se)
    for g in range(4):
        o_ref[NR * g:NR * (g + 1), :] = out[:, C * g:C * (g + 1)]


_tail = pl.pallas_call(
    _tail_body,
    out_shape=jax.ShapeDtypeStruct((N, C), jnp.float32),
)


def _bdiag4(w):
    return jax.scipy.linalg.block_diag(w, w, w, w)


def kernel(x, edge_index, W1, b1, W2, b2, g1, be1, W3, b3, W4, b4, g2, be2,
           Wf1, bf1, Wf2, bf2):
    er = edge_index.reshape(2, NCHT, CHUNK)
    q = ((er >= NR).astype(jnp.int32)
         + (er >= 2 * NR).astype(jnp.int32)
         + (er >= 3 * NR).astype(jnp.int32))
    edges = er * 4 - (4 * NR - 1) * q

    W2b, W3b, W4b, Wf1b = _bdiag4(W2), _bdiag4(W3), _bdiag4(W4), _bdiag4(Wf1)
    Wf2b = _bdiag4(Wf2)
    b1t, b2t, b3t, b4t = (jnp.tile(v, 4).reshape(1, 4 * D)
                          for v in (b1, b2, b3, b4))
    g1t, be1t, g2t, be2t = (jnp.tile(v, 4).reshape(1, 4 * D)
                            for v in (g1, be1, g2, be2))
    bf1t = jnp.tile(bf1, 4).reshape(1, 4 * D)
    bf2t = jnp.tile(bf2, 4).reshape(1, 4 * C)

    y1 = _proj(x, W1)
    parts1 = _seg_sum(y1.reshape(N, D), edges)
    y2 = _mid(y1, parts1.reshape(2, NPAD // 4, 4 * D), b1t, W2b, b2t, g1t,
              be1t, W3b)
    parts2 = _seg_sum(y2.reshape(N, D), edges)
    return _tail(y2, parts2.reshape(2, NPAD // 4, 4 * D), b3t, W4b, b4t, g2t,
                 be2t, Wf1b, bf1t, Wf2b, bf2t)


# speedup vs baseline: 1.0400x; 1.0400x over previous
import functools

import jax
import jax.numpy as jnp
from jax import lax
from jax.experimental import pallas as pl
from jax.experimental.pallas import tpu as pltpu
from jax.experimental.pallas import tpu_sc as plsc

N = 10000
E = 320000
F = 128
D = 32
C = 16

NW = 32
CHUNK = 128
NCHT = E // CHUNK
T1 = 1240
NCH0 = 79
NCH1 = 78
NPAD = 10112
RPT = NPAD // 16


RING = 6


def _seg_kernel_body(y_hbm, edge_hbm, out_hbm,
                     src_v, dst_v, rows, stage, acc, sem, scsem, sem_i):
    c = lax.axis_index("c")
    s = lax.axis_index("s")
    base = jnp.where(c == 0, T1 + s * 78 + jnp.maximum(s - 4, 0),
                     s * 77 + jnp.maximum(s - 8, 0))
    nch = jnp.where(c == 0, 78 + (s >= 4).astype(jnp.int32),
                    77 + (s >= 8).astype(jnp.int32))

    @pl.when(c == 0)
    def _():
        pltpu.async_copy(edge_hbm.at[0, pl.ds(base, NCH0)],
                         src_v.at[pl.ds(0, NCH0)], sem_i)
        pltpu.async_copy(edge_hbm.at[1, pl.ds(base, NCH0)],
                         dst_v.at[pl.ds(0, NCH0)], sem_i)

    @pl.when(c == 1)
    def _():
        pltpu.async_copy(edge_hbm.at[0, pl.ds(base, NCH1)],
                         src_v.at[pl.ds(0, NCH1)], sem_i)
        pltpu.async_copy(edge_hbm.at[1, pl.ds(base, NCH1)],
                         dst_v.at[pl.ds(0, NCH1)], sem_i)

    zero = jnp.zeros((16,), jnp.float32)

    def zrow(i, _):
        stage[i, pl.ds(0, 16)] = zero
        stage[i, pl.ds(16, 16)] = zero
        return 0

    lax.fori_loop(0, RPT, zrow, 0)
    pltpu.sync_copy(stage, acc.at[pl.ds(s * RPT, RPT)])

    @pl.when(c == 0)
    def _():
        pltpu.make_async_copy(edge_hbm.at[0, pl.ds(0, NCH0)],
                              src_v.at[pl.ds(0, NCH0)], sem_i).wait()
        pltpu.make_async_copy(edge_hbm.at[1, pl.ds(0, NCH0)],
                              dst_v.at[pl.ds(0, NCH0)], sem_i).wait()

    @pl.when(c == 1)
    def _():
        pltpu.make_async_copy(edge_hbm.at[0, pl.ds(0, NCH1)],
                              src_v.at[pl.ds(0, NCH1)], sem_i).wait()
        pltpu.make_async_copy(edge_hbm.at[1, pl.ds(0, NCH1)],
                              dst_v.at[pl.ds(0, NCH1)], sem_i).wait()

    plsc.subcore_barrier()

    for k in range(RING - 1):
        @pl.when(nch > k)
        def _(k=k):
            pltpu.async_copy(y_hbm.at[src_v.at[k]], rows.at[k], sem.at[k])

    def step(j, _):
        m = j % RING

        @pl.when(j + RING - 1 < nch)
        def _():
            m2 = (j + RING - 1) % RING

            @pl.when(j >= 1)
            def _():
                pltpu.make_async_copy(rows.at[m2], acc.at[dst_v.at[0]],
                                      scsem.at[m2]).wait()

            pltpu.async_copy(y_hbm.at[src_v.at[j + RING - 1]],
                             rows.at[m2], sem.at[m2])

        pltpu.make_async_copy(y_hbm.at[src_v.at[0]], rows.at[m], sem.at[m]).wait()
        pltpu.async_copy(rows.at[m], acc.at[dst_v.at[j]], scsem.at[m], add=True)
        return 0

    lax.fori_loop(0, nch, step, 0)

    for k in range(RING):
        pltpu.make_async_copy(rows.at[k], acc.at[dst_v.at[0]],
                              scsem.at[k]).wait()
    plsc.subcore_barrier()

    pltpu.sync_copy(acc.at[pl.ds(s * RPT, RPT)], stage)
    pltpu.sync_copy(stage, out_hbm.at[c, pl.ds(s * RPT, RPT)])


_seg_sum = pl.kernel(
    _seg_kernel_body,
    out_type=jax.ShapeDtypeStruct((2, NPAD, D), jnp.float32),
    mesh=plsc.VectorSubcoreMesh(core_axis_name="c", subcore_axis_name="s"),
    scratch_types=[
        pltpu.VMEM((NCH0, CHUNK), jnp.int32),
        pltpu.VMEM((NCH0, CHUNK), jnp.int32),
        pltpu.VMEM((RING, CHUNK, D), jnp.float32),
        pltpu.VMEM((RPT, D), jnp.float32),
        pltpu.VMEM_SHARED((NPAD, D), jnp.float32),
        pltpu.SemaphoreType.DMA((RING,)),
        pltpu.SemaphoreType.DMA((RING,)),
        pltpu.SemaphoreType.DMA,
    ],
    compiler_params=pltpu.CompilerParams(use_tc_tiling_on_sc=False),
)


NR = N // 4


def _mm_body(x_ref, w_ref, o_ref):
    for g in range(4):
        o_ref[:, D * g:D * (g + 1)] = jnp.dot(
            x_ref[NR * g:NR * (g + 1), :], w_ref[...],
            preferred_element_type=jnp.float32)


_proj = pl.pallas_call(
    _mm_body,
    out_shape=jax.ShapeDtypeStruct((NR, 4 * D), jnp.float32),
)


def _fold_stats(r):
    cs = jnp.sum(r, axis=0, keepdims=True)
    css = jnp.sum(r * r, axis=0, keepdims=True)
    s32 = cs[:, 0:32] + cs[:, 32:64] + cs[:, 64:96] + cs[:, 96:128]
    ss32 = css[:, 0:32] + css[:, 32:64] + css[:, 64:96] + css[:, 96:128]
    m = jnp.concatenate([s32] * 4, axis=1) * (1.0 / N)
    ms2 = jnp.concatenate([ss32] * 4, axis=1) * (1.0 / N)
    return m, ms2


def _bn128(r, g, b):
    m, ms2 = _fold_stats(r)
    v = ms2 - m * m
    return (r - m) * jax.lax.rsqrt(v + 1e-5) * g + b


def _gin128(y, parts, b_a, w_b, b_b):
    agg = parts[0, :NR, :] + parts[1, :NR, :]
    z = jnp.maximum(y + agg + b_a, 0.0)
    return jnp.dot(z, w_b, preferred_element_type=jnp.float32) + b_b


def _mid_body(y_ref, parts_ref, b1_ref, w2_ref, b2_ref, g1_ref, be1_ref,
              w3_ref, o_ref):
    h = jnp.maximum(_gin128(y_ref[...], parts_ref[...], b1_ref[...],
                            w2_ref[...], b2_ref[...]), 0.0)
    h = _bn128(h, g1_ref[...], be1_ref[...])
    o_ref[...] = jnp.dot(h, w3_ref[...], preferred_element_type=jnp.float32)


_mid = pl.pallas_call(
    _mid_body,
    out_shape=jax.ShapeDtypeStruct((NR, 4 * D), jnp.float32),
)


def _tail_body(y_ref, parts_ref, b3_ref, w4_ref, b4_ref, g2_ref, be2_ref,
               wf1_ref, bf1_ref, wf2_ref, bf2_ref, o_ref):
    h = jnp.maximum(_gin128(y_ref[...], parts_ref[...], b3_ref[...],
                            w4_ref[...], b4_ref[...]), 0.0)
    h = _bn128(h, g2_ref[...], be2_ref[...])
    f = jnp.maximum(
        jnp.dot(h, wf1_ref[...], preferred_element_type=jnp.float32)
        + bf1_ref[...], 0.0)
    logits = jnp.dot(f, wf2_ref[...], preferred_element_type=jnp.float32) \
        + bf2_ref[...]
    r64 = jax.lax.broadcasted_iota(jnp.int32, (4 * C, 4 * C), 0) // C
    c64 = jax.lax.broadcasted_iota(jnp.int32, (4 * C, 4 * C), 1) // C
    ones_blk = (r64 == c64).astype(jnp.float32)
    se = jnp.dot(jnp.exp(logits), ones_blk, preferred_element_type=jnp.float32)
    o_ref[...] = logits - jnp.log(se)


_tail = pl.pallas_call(
    _tail_body,
    out_shape=jax.ShapeDtypeStruct((NR, 4 * C), jnp.float32),
)


def _bdiag4(w):
    return jax.scipy.linalg.block_diag(w, w, w, w)


def kernel(x, edge_index, W1, b1, W2, b2, g1, be1, W3, b3, W4, b4, g2, be2,
           Wf1, bf1, Wf2, bf2):
    er = edge_index.reshape(2, NCHT, CHUNK)
    q = ((er >= NR).astype(jnp.int32)
         + (er >= 2 * NR).astype(jnp.int32)
         + (er >= 3 * NR).astype(jnp.int32))
    edges = er * 4 - (4 * NR - 1) * q

    W2b, W3b, W4b, Wf1b = _bdiag4(W2), _bdiag4(W3), _bdiag4(W4), _bdiag4(Wf1)
    Wf2b = _bdiag4(Wf2)
    b1t, b2t, b3t, b4t = (jnp.tile(v, 4).reshape(1, 4 * D)
                          for v in (b1, b2, b3, b4))
    g1t, be1t, g2t, be2t = (jnp.tile(v, 4).reshape(1, 4 * D)
                            for v in (g1, be1, g2, be2))
    bf1t = jnp.tile(bf1, 4).reshape(1, 4 * D)
    bf2t = jnp.tile(bf2, 4).reshape(1, 4 * C)

    y1 = _proj(x, W1)
    parts1 = _seg_sum(y1.reshape(N, D), edges)
    y2 = _mid(y1, parts1.reshape(2, NPAD // 4, 4 * D), b1t, W2b, b2t, g1t,
              be1t, W3b)
    parts2 = _seg_sum(y2.reshape(N, D), edges)
    out = _tail(y2, parts2.reshape(2, NPAD // 4, 4 * D), b3t, W4b, b4t, g2t,
                be2t, Wf1b, bf1t, Wf2b, bf2t)
    return jnp.swapaxes(out.reshape(NR, 4, C), 0, 1).reshape(N, C)
